# Initial kernel scaffold; baseline (speedup 1.0000x reference)
import jax
import jax.numpy as jnp
from jax.experimental import pallas as pl


def kernel(theta_mu, log_theta_s, z_w, z_b, log_z_s, eps_theta, eps_z):
    raise NotImplementedError("write your pallas kernel here")


# speedup vs baseline: 1.0087x; 1.0087x over previous
import jax
import jax.numpy as jnp
from jax.experimental import pallas as pl

_BLOCK_N = 4000


def _ew_kernel(theta_mu_ref, log_theta_s_ref, eps_theta_ref,
               z_w_ref, z_b_ref, log_z_s_ref, eps_z_ref, out_ref):
    theta = theta_mu_ref[:] + jnp.exp(log_theta_s_ref[:]) * eps_theta_ref[:]
    out_ref[:] = (z_w_ref[:] * theta + z_b_ref[:]
                  + jnp.exp(log_z_s_ref[:]) * eps_z_ref[:])


def kernel(theta_mu, log_theta_s, z_w, z_b, log_z_s, eps_theta, eps_z):
    n, j = z_w.shape
    block_n = _BLOCK_N if n % _BLOCK_N == 0 else n
    grid = (n // block_n,)

    small = pl.BlockSpec((1, j), lambda i: (0, 0))
    big = pl.BlockSpec((block_n, j), lambda i: (i, 0))

    return pl.pallas_call(
        _ew_kernel,
        grid=grid,
        in_specs=[small, small, small, big, big, big, big],
        out_specs=big,
        out_shape=jax.ShapeDtypeStruct((n, j), z_w.dtype),
    )(theta_mu.reshape(1, j), log_theta_s.reshape(1, j),
      eps_theta.reshape(1, j), z_w, z_b, log_z_s, eps_z)
